# baseline (speedup 1.0000x reference)
import functools

import jax
import jax.numpy as jnp
from jax import lax
from jax.experimental import pallas as pl
from jax.experimental.pallas import tpu as pltpu
from jax.experimental.pallas import tpu_sc as plsc

_THRESHOLD = 0.8
_NUM_CLASSES = 80
_NPAD = 5120
_NVEC = _NPAD // 16
_NC = 1
_NS = 16
_NT = _NC * _NS
_SLICE = _NPAD // _NS

_mesh = plsc.VectorSubcoreMesh(
    core_axis_name="c", subcore_axis_name="s",
    num_cores=_NC, num_subcores=_NS)


def _sc_body(x1h, y1h, x2h, y2h, cath, outh0,
             x1v, y1v, x2v, y2v, catv,
             midx, mx1, my1, mx2, my2, mar, keepm,
             keep_full, accv, rowv, shared):
    cid = lax.axis_index("c")
    sid = lax.axis_index("s")
    gwid = cid * _NS + sid

    pltpu.sync_copy(x1h, x1v)
    pltpu.sync_copy(y1h, y1v)
    pltpu.sync_copy(x2h, x2v)
    pltpu.sync_copy(y2h, y2v)
    pltpu.sync_copy(cath, catv)

    zeros16 = jnp.zeros((16,), jnp.float32)
    iota16 = lax.iota(jnp.int32, 16)

    def zero_body(u, _):
        keep_full[pl.ds(pl.multiple_of(u * 16, 16), 16)] = zeros16
        return 0
    lax.fori_loop(0, _NVEC, zero_body, 0)

    def process(c):
        def scan_body(v, cnt):
            off = pl.multiple_of(v * 16, 16)
            c16 = catv[pl.ds(off, 16)]
            m = c16 == c
            mi = m.astype(jnp.int32)
            pc = plsc.cumsum(mi)
            pos = cnt + pc - 1
            plsc.store_scatter(midx, [pos], off + iota16, mask=m)
            return cnt + jnp.sum(mi)
        num = lax.fori_loop(0, _NVEC, scan_body, jnp.int32(0))
        nv = (num + 15) // 16

        def gather_body(u, _):
            off = pl.multiple_of(u * 16, 16)
            valid = (off + iota16) < num
            idx16 = jnp.where(valid, midx[pl.ds(off, 16)], 0)
            gx1 = plsc.load_gather(x1v, [idx16])
            gy1 = plsc.load_gather(y1v, [idx16])
            gx2 = plsc.load_gather(x2v, [idx16])
            gy2 = plsc.load_gather(y2v, [idx16])
            mx1[pl.ds(off, 16)] = gx1
            my1[pl.ds(off, 16)] = gy1
            mx2[pl.ds(off, 16)] = gx2
            my2[pl.ds(off, 16)] = gy2
            mar[pl.ds(off, 16)] = (gx2 - gx1) * (gy2 - gy1)
            return 0
        lax.fori_loop(0, nv, gather_body, 0)

        def rowchunk_body(t, _):
            roff = pl.multiple_of(t * 16, 16)
            vx1 = mx1[pl.ds(roff, 16)]
            vy1 = my1[pl.ds(roff, 16)]
            vx2 = mx2[pl.ds(roff, 16)]
            vy2 = my2[pl.ds(roff, 16)]
            var = mar[pl.ds(roff, 16)]
            keep16 = zeros16
            for lane in range(16):
                i = roff + lane
                rx1 = vx1[lane]
                ry1 = vy1[lane]
                rx2 = vx2[lane]
                ry2 = vy2[lane]
                ra = var[lane]

                def col_body(u, acc):
                    off = pl.multiple_of(u * 16, 16)
                    lanes = off + iota16
                    cx1 = mx1[pl.ds(off, 16)]
                    cy1 = my1[pl.ds(off, 16)]
                    cx2 = mx2[pl.ds(off, 16)]
                    cy2 = my2[pl.ds(off, 16)]
                    ca = mar[pl.ds(off, 16)]
                    ok = ((lanes < num) & (lanes != i)
                          & (cx1 >= rx1) & (cy1 >= ry1)
                          & (cx2 <= rx2) & (cy2 <= ry2))
                    return acc + jnp.where(ok, ca, 0.0)

                acc = lax.fori_loop(0, nv, col_body, zeros16)
                s = jnp.sum(acc)
                k = jnp.where(s <= _THRESHOLD * (ra + 1e-9),
                              jnp.float32(1.0), jnp.float32(0.0))
                keep16 = jnp.where(iota16 == lane, k, keep16)
            keepm[pl.ds(roff, 16)] = keep16
            return 0
        lax.fori_loop(0, nv, rowchunk_body, 0)

        def scat_body(u, _):
            off = pl.multiple_of(u * 16, 16)
            valid = (off + iota16) < num
            idx16 = midx[pl.ds(off, 16)]
            k16 = keepm[pl.ds(off, 16)]
            plsc.store_scatter(keep_full, [idx16], k16, mask=valid)
            return 0
        lax.fori_loop(0, nv, scat_body, 0)

    for kslot in range(-(-_NUM_CLASSES // _NT)):
        c = gwid + _NT * kslot

        @pl.when(c < _NUM_CLASSES)
        def _():
            process(c)

    pltpu.sync_copy(keep_full, shared.at[pl.ds(sid * _NPAD, _NPAD)])
    plsc.subcore_barrier()

    base = sid * _SLICE

    def acc_zero(u, _):
        accv[pl.ds(pl.multiple_of(u * 16, 16), 16)] = zeros16
        return 0
    lax.fori_loop(0, _SLICE // 16, acc_zero, 0)

    for r in range(_NS):
        pltpu.sync_copy(shared.at[pl.ds(r * _NPAD + base, _SLICE)], rowv)

        def add_body(u, _):
            o = pl.multiple_of(u * 16, 16)
            accv[pl.ds(o, 16)] = accv[pl.ds(o, 16)] + rowv[pl.ds(o, 16)]
            return 0
        lax.fori_loop(0, _SLICE // 16, add_body, 0)

    pltpu.sync_copy(accv, outh0.at[pl.ds(base, _SLICE)])


_sc_filter = functools.partial(
    pl.kernel,
    out_type=jax.ShapeDtypeStruct((_NPAD,), jnp.float32),
    mesh=_mesh,
    compiler_params=pltpu.CompilerParams(needs_layout_passes=False),
    scratch_types=[
        pltpu.VMEM((_NPAD,), jnp.float32),
        pltpu.VMEM((_NPAD,), jnp.float32),
        pltpu.VMEM((_NPAD,), jnp.float32),
        pltpu.VMEM((_NPAD,), jnp.float32),
        pltpu.VMEM((_NPAD,), jnp.int32),
        pltpu.VMEM((_NPAD,), jnp.int32),
        pltpu.VMEM((_NPAD,), jnp.float32),
        pltpu.VMEM((_NPAD,), jnp.float32),
        pltpu.VMEM((_NPAD,), jnp.float32),
        pltpu.VMEM((_NPAD,), jnp.float32),
        pltpu.VMEM((_NPAD,), jnp.float32),
        pltpu.VMEM((_NPAD,), jnp.float32),
        pltpu.VMEM((_NPAD,), jnp.float32),
        pltpu.VMEM((_SLICE,), jnp.float32),
        pltpu.VMEM((_SLICE,), jnp.float32),
        pltpu.VMEM_SHARED((_NS * _NPAD,), jnp.float32),
    ],
)(_sc_body)


def kernel(boxes, scores, category_ids):
    n = boxes.shape[0]
    cat = category_ids.astype(jnp.int32)
    pad = _NPAD - n
    bp = jnp.pad(boxes, ((0, pad), (0, 0)))
    cp = jnp.pad(cat, (0, pad), constant_values=-1)
    x1 = bp[:, 0]
    y1 = bp[:, 1]
    x2 = bp[:, 2]
    y2 = bp[:, 3]

    p0 = _sc_filter(x1, y1, x2, y2, cp)
    keep = p0[:n]
    box5 = jnp.concatenate([boxes, scores[:, None]], axis=1)
    return box5 * keep[:, None]


# speedup vs baseline: 1.1653x; 1.1653x over previous
import functools

import jax
import jax.numpy as jnp
from jax import lax
from jax.experimental import pallas as pl
from jax.experimental.pallas import tpu as pltpu
from jax.experimental.pallas import tpu_sc as plsc

_THRESHOLD = 0.8
_NUM_CLASSES = 80
_NPAD = 5120
_NVEC = _NPAD // 16
_NC = 2
_NS = 16
_NT = _NC * _NS
_SLICE = _NPAD // _NS

_mesh = plsc.VectorSubcoreMesh(
    core_axis_name="c", subcore_axis_name="s",
    num_cores=_NC, num_subcores=_NS)


def _sc_body(x1h, y1h, x2h, y2h, cath, outh0, outh1,
             x1v, y1v, x2v, y2v, catv,
             midx, mx1, my1, mx2, my2, mar, keepm,
             keep_full, accv, rowv, shared):
    cid = lax.axis_index("c")
    sid = lax.axis_index("s")
    gwid = cid * _NS + sid

    pltpu.sync_copy(x1h, x1v)
    pltpu.sync_copy(y1h, y1v)
    pltpu.sync_copy(x2h, x2v)
    pltpu.sync_copy(y2h, y2v)
    pltpu.sync_copy(cath, catv)

    zeros16 = jnp.zeros((16,), jnp.float32)
    iota16 = lax.iota(jnp.int32, 16)

    def zero_body(u, _):
        keep_full[pl.ds(pl.multiple_of(u * 16, 16), 16)] = zeros16
        return 0
    lax.fori_loop(0, _NVEC, zero_body, 0)

    def process(c):
        def scan_body(v, cnt):
            off = pl.multiple_of(v * 16, 16)
            c16 = catv[pl.ds(off, 16)]
            m = c16 == c
            plsc.store_compressed(midx.at[pl.ds(cnt, 16)], off + iota16, mask=m)
            return cnt + plsc.all_reduce_population_count(m)[0]
        num = lax.fori_loop(0, _NVEC, scan_body, jnp.int32(0))
        nv = (num + 15) // 16

        def gather_body(u, _):
            off = pl.multiple_of(u * 16, 16)
            valid = (off + iota16) < num
            idx16 = jnp.where(valid, midx[pl.ds(off, 16)], 0)
            gx1 = plsc.load_gather(x1v, [idx16])
            gy1 = plsc.load_gather(y1v, [idx16])
            gx2 = plsc.load_gather(x2v, [idx16])
            gy2 = plsc.load_gather(y2v, [idx16])
            mx1[pl.ds(off, 16)] = gx1
            my1[pl.ds(off, 16)] = gy1
            mx2[pl.ds(off, 16)] = gx2
            my2[pl.ds(off, 16)] = gy2
            mar[pl.ds(off, 16)] = (gx2 - gx1) * (gy2 - gy1)
            return 0
        lax.fori_loop(0, nv, gather_body, 0)

        def rowchunk_body(t, _):
            roff = pl.multiple_of(t * 16, 16)
            rx1 = mx1[pl.ds(roff, 16)]
            ry1 = my1[pl.ds(roff, 16)]
            rx2 = mx2[pl.ds(roff, 16)]
            ry2 = my2[pl.ds(roff, 16)]
            rar = mar[pl.ds(roff, 16)]
            rlanes = roff + iota16

            def colchunk_body(u, acc):
                off = pl.multiple_of(u * 16, 16)
                cx1 = mx1[pl.ds(off, 16)]
                cy1 = my1[pl.ds(off, 16)]
                cx2 = mx2[pl.ds(off, 16)]
                cy2 = my2[pl.ds(off, 16)]
                car = mar[pl.ds(off, 16)]
                for lane in range(16):
                    j = off + lane
                    ja = jnp.where(j < num, car[lane], jnp.float32(0.0))
                    ok = ((cx1[lane] >= rx1) & (cy1[lane] >= ry1)
                          & (cx2[lane] <= rx2) & (cy2[lane] <= ry2)
                          & (rlanes != j))
                    acc = acc + jnp.where(ok, ja, jnp.float32(0.0))
                return acc

            acc = lax.fori_loop(0, nv, colchunk_body, zeros16)
            keep16 = jnp.where(acc <= _THRESHOLD * (rar + 1e-9),
                               jnp.float32(1.0), jnp.float32(0.0))
            keepm[pl.ds(roff, 16)] = keep16
            return 0
        lax.fori_loop(0, nv, rowchunk_body, 0)

        def scat_body(u, _):
            off = pl.multiple_of(u * 16, 16)
            valid = (off + iota16) < num
            idx16 = midx[pl.ds(off, 16)]
            k16 = keepm[pl.ds(off, 16)]
            plsc.store_scatter(keep_full, [idx16], k16, mask=valid)
            return 0
        lax.fori_loop(0, nv, scat_body, 0)

    for kslot in range(3):
        c = gwid + _NT * kslot

        @pl.when(c < _NUM_CLASSES)
        def _():
            process(c)

    pltpu.sync_copy(keep_full, shared.at[pl.ds(sid * _NPAD, _NPAD)])
    plsc.subcore_barrier()

    base = sid * _SLICE

    def acc_zero(u, _):
        accv[pl.ds(pl.multiple_of(u * 16, 16), 16)] = zeros16
        return 0
    lax.fori_loop(0, _SLICE // 16, acc_zero, 0)

    for r in range(_NS):
        pltpu.sync_copy(shared.at[pl.ds(r * _NPAD + base, _SLICE)], rowv)

        def add_body(u, _):
            o = pl.multiple_of(u * 16, 16)
            accv[pl.ds(o, 16)] = accv[pl.ds(o, 16)] + rowv[pl.ds(o, 16)]
            return 0
        lax.fori_loop(0, _SLICE // 16, add_body, 0)

    @pl.when(cid == 0)
    def _():
        pltpu.sync_copy(accv, outh0.at[pl.ds(base, _SLICE)])

    @pl.when(cid == 1)
    def _():
        pltpu.sync_copy(accv, outh1.at[pl.ds(base, _SLICE)])


_sc_filter = functools.partial(
    pl.kernel,
    out_type=[jax.ShapeDtypeStruct((_NPAD,), jnp.float32),
              jax.ShapeDtypeStruct((_NPAD,), jnp.float32)],
    mesh=_mesh,
    compiler_params=pltpu.CompilerParams(needs_layout_passes=False),
    scratch_types=[
        pltpu.VMEM((_NPAD,), jnp.float32),
        pltpu.VMEM((_NPAD,), jnp.float32),
        pltpu.VMEM((_NPAD,), jnp.float32),
        pltpu.VMEM((_NPAD,), jnp.float32),
        pltpu.VMEM((_NPAD,), jnp.int32),
        pltpu.VMEM((_NPAD,), jnp.int32),
        pltpu.VMEM((_NPAD,), jnp.float32),
        pltpu.VMEM((_NPAD,), jnp.float32),
        pltpu.VMEM((_NPAD,), jnp.float32),
        pltpu.VMEM((_NPAD,), jnp.float32),
        pltpu.VMEM((_NPAD,), jnp.float32),
        pltpu.VMEM((_NPAD,), jnp.float32),
        pltpu.VMEM((_NPAD,), jnp.float32),
        pltpu.VMEM((_SLICE,), jnp.float32),
        pltpu.VMEM((_SLICE,), jnp.float32),
        pltpu.VMEM_SHARED((_NS * _NPAD,), jnp.float32),
    ],
)(_sc_body)


def kernel(boxes, scores, category_ids):
    n = boxes.shape[0]
    cat = category_ids.astype(jnp.int32)
    pad = _NPAD - n
    bp = jnp.pad(boxes, ((0, pad), (0, 0)))
    cp = jnp.pad(cat, (0, pad), constant_values=-1)
    x1 = bp[:, 0]
    y1 = bp[:, 1]
    x2 = bp[:, 2]
    y2 = bp[:, 3]

    p0, p1 = _sc_filter(x1, y1, x2, y2, cp)
    keep = (p0 + p1)[:n]
    box5 = jnp.concatenate([boxes, scores[:, None]], axis=1)
    return box5 * keep[:, None]


# speedup vs baseline: 1.4932x; 1.2814x over previous
import functools

import jax
import jax.numpy as jnp
import numpy as np
from jax import lax
from jax.experimental import pallas as pl
from jax.experimental.pallas import tpu as pltpu
from jax.experimental.pallas import tpu_sc as plsc

_THRESHOLD = 0.8
_NUM_CLASSES = 80
_NPAD = 5120
_NVEC = _NPAD // 16
_NC = 2
_NS = 16
_NT = _NC * _NS
_SLICE = _NPAD // _NS
_KSLOTS = -(-_NUM_CLASSES // _NT)

_mesh = plsc.VectorSubcoreMesh(
    core_axis_name="c", subcore_axis_name="s",
    num_cores=_NC, num_subcores=_NS)

def _sc_body(x1h, y1h, x2h, y2h, cath, outh0, outh1,
             x1v, y1v, x2v, y2v, catv,
             midx0, midx1, midx2, mx1, my1, mx2, my2, mar, keepm,
             keep_full, accv, rowv, shared):
    cid = lax.axis_index("c")
    sid = lax.axis_index("s")
    gwid = cid * _NS + sid

    pltpu.sync_copy(x1h, x1v)
    pltpu.sync_copy(y1h, y1v)
    pltpu.sync_copy(x2h, x2v)
    pltpu.sync_copy(y2h, y2v)
    pltpu.sync_copy(cath, catv)

    zeros16 = jnp.zeros((16,), jnp.float32)
    iota16 = lax.iota(jnp.int32, 16)

    def zero_body(u, _):
        keep_full[pl.ds(pl.multiple_of(u * 16, 16), 16)] = zeros16
        return 0
    lax.fori_loop(0, _NVEC, zero_body, 0)

    cls = [gwid + _NT * k for k in range(_KSLOTS)]
    midxs = [midx0, midx1, midx2]

    def scan_body(v, cnts):
        off = pl.multiple_of(v * 16, 16)
        c16 = catv[pl.ds(off, 16)]
        gidx = off + iota16
        out = []
        for k in range(_KSLOTS):
            m = c16 == cls[k]
            plsc.store_compressed(midxs[k].at[pl.ds(cnts[k], 16)], gidx,
                                  mask=m)
            out.append(cnts[k] + plsc.all_reduce_population_count(m)[0])
        return tuple(out)
    nums = lax.fori_loop(0, _NVEC, scan_body,
                         (jnp.int32(0),) * _KSLOTS)

    def process(midx, num):
        nv = (num + 15) // 16

        def gather_body(u, _):
            off = pl.multiple_of(u * 16, 16)
            valid = (off + iota16) < num
            idx16 = jnp.where(valid, midx[pl.ds(off, 16)], 0)
            gx1 = plsc.load_gather(x1v, [idx16])
            gy1 = plsc.load_gather(y1v, [idx16])
            gx2 = plsc.load_gather(x2v, [idx16])
            gy2 = plsc.load_gather(y2v, [idx16])
            mx1[pl.ds(off, 16)] = gx1
            my1[pl.ds(off, 16)] = gy1
            mx2[pl.ds(off, 16)] = gx2
            my2[pl.ds(off, 16)] = gy2
            mar[pl.ds(off, 16)] = (gx2 - gx1) * (gy2 - gy1)
            return 0
        lax.fori_loop(0, nv, gather_body, 0)

        def rowchunk_body(t, _):
            roff = pl.multiple_of(t * 16, 16)
            rx1 = mx1[pl.ds(roff, 16)]
            ry1 = my1[pl.ds(roff, 16)]
            rx2 = mx2[pl.ds(roff, 16)]
            ry2 = my2[pl.ds(roff, 16)]
            rar = mar[pl.ds(roff, 16)]
            rlanes = roff + iota16

            def colchunk_body(u, acc):
                off = pl.multiple_of(u * 16, 16)
                cx1 = mx1[pl.ds(off, 16)]
                cy1 = my1[pl.ds(off, 16)]
                cx2 = mx2[pl.ds(off, 16)]
                cy2 = my2[pl.ds(off, 16)]
                car = mar[pl.ds(off, 16)]
                clanes = off + iota16
                car_m = jnp.where(clanes < num, car, jnp.float32(0.0))
                for r in range(16):
                    if r == 0:
                        gx1, gy1, gx2, gy2 = cx1, cy1, cx2, cy2
                        gca, cvec = car_m, clanes
                    else:
                        p = (iota16 + r) & 15
                        gx1 = cx1.at[p].get(mode="promise_in_bounds")
                        gy1 = cy1.at[p].get(mode="promise_in_bounds")
                        gx2 = cx2.at[p].get(mode="promise_in_bounds")
                        gy2 = cy2.at[p].get(mode="promise_in_bounds")
                        gca = car_m.at[p].get(mode="promise_in_bounds")
                        cvec = off + p
                    ok = ((gx1 >= rx1) & (gy1 >= ry1)
                          & (gx2 <= rx2) & (gy2 <= ry2)
                          & (rlanes != cvec))
                    acc = acc + jnp.where(ok, gca, jnp.float32(0.0))
                return acc

            acc = lax.fori_loop(0, nv, colchunk_body, zeros16)
            keep16 = jnp.where(acc <= _THRESHOLD * (rar + 1e-9),
                               jnp.float32(1.0), jnp.float32(0.0))
            keepm[pl.ds(roff, 16)] = keep16
            return 0
        lax.fori_loop(0, nv, rowchunk_body, 0)

        def scat_body(u, _):
            off = pl.multiple_of(u * 16, 16)
            valid = (off + iota16) < num
            idx16 = midx[pl.ds(off, 16)]
            k16 = keepm[pl.ds(off, 16)]
            plsc.store_scatter(keep_full, [idx16], k16, mask=valid)
            return 0
        lax.fori_loop(0, nv, scat_body, 0)

    for k in range(_KSLOTS):
        process(midxs[k], nums[k])

    pltpu.sync_copy(keep_full, shared.at[pl.ds(sid * _NPAD, _NPAD)])
    plsc.subcore_barrier()

    base = sid * _SLICE

    def acc_zero(u, _):
        accv[pl.ds(pl.multiple_of(u * 16, 16), 16)] = zeros16
        return 0
    lax.fori_loop(0, _SLICE // 16, acc_zero, 0)

    for r in range(_NS):
        pltpu.sync_copy(shared.at[pl.ds(r * _NPAD + base, _SLICE)], rowv)

        def add_body(u, _):
            o = pl.multiple_of(u * 16, 16)
            accv[pl.ds(o, 16)] = accv[pl.ds(o, 16)] + rowv[pl.ds(o, 16)]
            return 0
        lax.fori_loop(0, _SLICE // 16, add_body, 0)

    @pl.when(cid == 0)
    def _():
        pltpu.sync_copy(accv, outh0.at[pl.ds(base, _SLICE)])

    @pl.when(cid == 1)
    def _():
        pltpu.sync_copy(accv, outh1.at[pl.ds(base, _SLICE)])


_sc_filter = functools.partial(
    pl.kernel,
    out_type=[jax.ShapeDtypeStruct((_NPAD,), jnp.float32),
              jax.ShapeDtypeStruct((_NPAD,), jnp.float32)],
    mesh=_mesh,
    compiler_params=pltpu.CompilerParams(needs_layout_passes=False),
    scratch_types=[
        pltpu.VMEM((_NPAD,), jnp.float32),
        pltpu.VMEM((_NPAD,), jnp.float32),
        pltpu.VMEM((_NPAD,), jnp.float32),
        pltpu.VMEM((_NPAD,), jnp.float32),
        pltpu.VMEM((_NPAD,), jnp.int32),
        pltpu.VMEM((_NPAD,), jnp.int32),
        pltpu.VMEM((_NPAD,), jnp.int32),
        pltpu.VMEM((_NPAD,), jnp.int32),
        pltpu.VMEM((_NPAD,), jnp.float32),
        pltpu.VMEM((_NPAD,), jnp.float32),
        pltpu.VMEM((_NPAD,), jnp.float32),
        pltpu.VMEM((_NPAD,), jnp.float32),
        pltpu.VMEM((_NPAD,), jnp.float32),
        pltpu.VMEM((_NPAD,), jnp.float32),
        pltpu.VMEM((_NPAD,), jnp.float32),
        pltpu.VMEM((_SLICE,), jnp.float32),
        pltpu.VMEM((_SLICE,), jnp.float32),
        pltpu.VMEM_SHARED((_NS * _NPAD,), jnp.float32),
    ],
)(_sc_body)


def kernel(boxes, scores, category_ids):
    n = boxes.shape[0]
    cat = category_ids.astype(jnp.int32)
    pad = _NPAD - n
    bp = jnp.pad(boxes, ((0, pad), (0, 0)))
    cp = jnp.pad(cat, (0, pad), constant_values=-1)
    x1 = bp[:, 0]
    y1 = bp[:, 1]
    x2 = bp[:, 2]
    y2 = bp[:, 3]

    p0, p1 = _sc_filter(x1, y1, x2, y2, cp)
    keep = (p0 + p1)[:n]
    box5 = jnp.concatenate([boxes, scores[:, None]], axis=1)
    return box5 * keep[:, None]
